# Initial kernel scaffold; baseline (speedup 1.0000x reference)
import jax
import jax.numpy as jnp
from jax.experimental import pallas as pl


def kernel(x, edge_index, edge_attr, batch, atom_emb, bond_emb, w_gate, w_noise):
    raise NotImplementedError("write your pallas kernel here")


# speedup vs baseline: 3.5611x; 3.5611x over previous
import functools

import jax
import jax.numpy as jnp
from jax import lax
from jax.experimental import pallas as pl
from jax.experimental.pallas import tpu as pltpu
from jax.experimental.pallas import tpu_sc as plsc

N = 10000
D = 256
NE = 16
K = 4
G = 512
ATOM_V = 100
NAF = 9
TAB = NAF * ATOM_V

NC = 2
NS = 16
NW = NC * NS
CPW = 320
NPAD = NW * CPW
GA = G + 8


NBLK = 8
BLK = NPAD // NBLK


def _logits_body(x_ref, emb_ref, w_ref, out_ref):
    xb = x_ref[...]
    vocab = lax.broadcasted_iota(jnp.int32, (BLK, ATOM_V), 1)
    oh = jnp.concatenate(
        [(xb[:, f:f + 1] == vocab).astype(jnp.bfloat16) for f in range(NAF)],
        axis=1)

    e = emb_ref[...]
    hi = e.astype(jnp.bfloat16)
    r1 = e - hi.astype(jnp.float32)
    mid = r1.astype(jnp.bfloat16)
    lo = (r1 - mid.astype(jnp.float32)).astype(jnp.bfloat16)

    h = jnp.dot(oh, hi, preferred_element_type=jnp.float32)
    h = h + jnp.dot(oh, mid, preferred_element_type=jnp.float32)
    h = h + jnp.dot(oh, lo, preferred_element_type=jnp.float32)

    hb = h.astype(jnp.bfloat16)
    wb = w_ref[...].astype(jnp.bfloat16)
    out_ref[...] = jnp.dot(hb, wb, preferred_element_type=jnp.float32)


def _node_logits(xpad, emb_flat, w_gate):
    return pl.pallas_call(
        _logits_body,
        grid=(NBLK,),
        in_specs=[
            pl.BlockSpec((BLK, 16), lambda i: (i, 0)),
            pl.BlockSpec((TAB, D), lambda i: (0, 0)),
            pl.BlockSpec((D, NE), lambda i: (0, 0)),
        ],
        out_specs=pl.BlockSpec((BLK, NE), lambda i: (i, 0)),
        out_shape=jax.ShapeDtypeStruct((NPAD, NE), jnp.float32),
    )(xpad, emb_flat, w_gate)


def _sc_body(log_hbm, scat_hbm, psum_hbm, pcnt_hbm,
             log_v, scat_v, acc_v, cacc_v, sem2):
    wid = lax.axis_index("s") * NC + lax.axis_index("c")
    lanes = lax.iota(jnp.int32, 16)
    zero = jnp.zeros((16,), jnp.float32)
    e0 = jnp.where(lanes == 0, 1.0, 0.0).astype(jnp.float32)

    cp_log = pltpu.async_copy(
        log_hbm.at[pl.ds(wid * CPW, CPW), :], log_v, sem2)
    cp_scat = pltpu.async_copy(
        scat_hbm.at[pl.ds(wid * CPW, CPW), :], scat_v, sem2)

    def _zero(i, _):
        acc_v[pl.ds(i * NE, NE)] = zero
        cacc_v[pl.ds(i * NE, NE)] = zero
        return 0
    lax.fori_loop(0, GA, _zero, 0)

    cp_log.wait()
    cp_scat.wait()

    def _node(n, _):
        r = log_v[n, :]
        sidx = scat_v[n, :]
        plsc.addupdate_scatter(acc_v, [sidx], r)
        plsc.addupdate_scatter(cacc_v, [sidx], e0)
        return 0
    lax.fori_loop(0, CPW, _node, 0)

    cp_a = pltpu.async_copy(acc_v, psum_hbm.at[wid], sem2)
    cp_b = pltpu.async_copy(cacc_v, pcnt_hbm.at[wid], sem2)
    cp_a.wait()
    cp_b.wait()


@functools.partial(
    pl.kernel,
    out_type=[
        jax.ShapeDtypeStruct((NW, GA * NE), jnp.float32),
        jax.ShapeDtypeStruct((NW, GA * NE), jnp.float32),
    ],
    mesh=plsc.VectorSubcoreMesh(core_axis_name="c", subcore_axis_name="s"),
    compiler_params=pltpu.CompilerParams(
        needs_layout_passes=False, use_tc_tiling_on_sc=False),
    scratch_types=[
        pltpu.VMEM((CPW, NE), jnp.float32),
        pltpu.VMEM((CPW, NE), jnp.int32),
        pltpu.VMEM((GA * NE,), jnp.float32),
        pltpu.VMEM((GA * NE,), jnp.float32),
        pltpu.SemaphoreType.DMA,
    ],
)
def _sc_segment_sum(log_hbm, scat_hbm, psum_hbm, pcnt_hbm,
                    log_v, scat_v, acc_v, cacc_v, sem2):
    _sc_body(log_hbm, scat_hbm, psum_hbm, pcnt_hbm,
             log_v, scat_v, acc_v, cacc_v, sem2)


def _cv_sq(v):
    m = jnp.mean(v)
    var = jnp.sum((v - m) ** 2) / (NE - 1)
    return var / (m * m + 1e-10)


def _final_body(psum_ref, pcnt_ref, gates_ref, load_ref, loss_ref):
    sums = jnp.sum(psum_ref[...], axis=0)[:G, :]
    counts = jnp.sum(pcnt_ref[...], axis=0)[:G, 0:1]
    logits = sums / jnp.maximum(counts, 1.0)

    neg = jnp.float32(-3.0e38)
    lane2d = lax.broadcasted_iota(jnp.int32, (G, NE), 1)
    work = logits
    gates = jnp.zeros((G, NE), jnp.float32)
    vals = []
    masks = []
    for _ in range(K):
        m = jnp.max(work, axis=1, keepdims=True)
        ism = work == m
        lo = jnp.min(jnp.where(ism, lane2d, NE), axis=1, keepdims=True)
        first = lane2d == lo
        vals.append(m)
        masks.append(first)
        work = jnp.where(first, neg, work)
    v0 = vals[0]
    exps = [jnp.exp(v - v0) for v in vals]
    denom = exps[0]
    for e in exps[1:]:
        denom = denom + e
    for e, msk in zip(exps, masks):
        gates = jnp.where(msk, e / denom, gates)
    gates_ref[...] = gates

    loadf = jnp.sum((gates > 0).astype(jnp.float32), axis=0, keepdims=True)
    load_ref[...] = loadf.astype(jnp.int32)
    importance = jnp.sum(gates, axis=0)
    loss = (_cv_sq(importance) + _cv_sq(jnp.sum(loadf, axis=0))) * 0.001
    loss_ref[...] = jnp.full((1, 1), loss, jnp.float32)


def _finalize(psum, pcnt):
    return pl.pallas_call(
        _final_body,
        out_shape=(
            jax.ShapeDtypeStruct((G, NE), jnp.float32),
            jax.ShapeDtypeStruct((1, NE), jnp.int32),
            jax.ShapeDtypeStruct((1, 1), jnp.float32),
        ),
    )(psum, pcnt)


def kernel(x, edge_index, edge_attr, batch, atom_emb, bond_emb, w_gate, w_noise):
    del edge_index, edge_attr, bond_emb, w_noise

    xpad = jnp.pad(x, ((0, NPAD - N), (0, 16 - NAF)))
    logits = _node_logits(xpad, atom_emb.reshape(TAB, D), w_gate)

    scat = jnp.pad(batch, (0, NPAD - N), constant_values=G)
    lanes = jnp.arange(NE, dtype=jnp.int32)[None, :]
    scat16 = (scat[:, None].astype(jnp.int32) * NE + lanes)

    psum, pcnt = _sc_segment_sum(logits, scat16)
    psum = psum.reshape(NW, GA, NE)
    pcnt = pcnt.reshape(NW, GA, NE)

    gates, load2d, loss2d = _finalize(psum, pcnt)
    return gates, load2d.reshape(NE), loss2d.reshape(())
